# baseline (speedup 1.0000x reference)
import jax
import jax.numpy as jnp
from jax import lax
from jax.experimental import pallas as pl
from jax.experimental.pallas import tpu as pltpu
from jax.experimental.pallas import tpu_sc as plsc

V = 100000
D = 64
B = 1024
C = 200

NC = 2
NS = 16
NW = NC * NS
NB = B // NW
IDX_PER_W = NB * C

TB = 32
EPS = 1e-8


def _query_body(ctx_hbm, miss_hbm, table_hbm, out_hbm,
                idx_v, rows0_v, rows1_v, midx_v, mrows_v, q_v, gsems, msem):
    c = lax.axis_index("c")
    s = lax.axis_index("s")
    w = s * NC + c
    base = w * NB

    pltpu.sync_copy(ctx_hbm.at[pl.ds(base * C, IDX_PER_W)], idx_v)
    pltpu.sync_copy(miss_hbm.at[pl.ds(base, NB)], midx_v)
    pltpu.async_copy(table_hbm.at[midx_v], mrows_v, msem).wait()

    inv_c = jnp.float32(1.0 / C)
    bufs = (rows0_v, rows1_v)

    def fire(i, buf):
        sem = gsems.at[i % 2]
        c0 = pltpu.async_copy(table_hbm.at[idx_v.at[pl.ds(i * C, 128)]],
                              buf.at[pl.ds(0, 128)], sem)
        c1 = pltpu.async_copy(table_hbm.at[idx_v.at[pl.ds(i * C + 128, C - 128)]],
                              buf.at[pl.ds(128, C - 128)], sem)
        return c0, c1

    pend = fire(0, bufs[0])
    zero = jnp.zeros((16,), jnp.float32)
    for i in range(NB):
        nxt = fire(i + 1, bufs[(i + 1) % 2]) if i + 1 < NB else None
        pend[0].wait()
        pend[1].wait()
        buf = bufs[i % 2]

        def red(j, accs, buf=buf):
            out = list(accs)
            for u in range(8):
                r = 8 * j + u
                p = u % 2
                for g in range(4):
                    k = g * 2 + p
                    out[k] = out[k] + buf[r, pl.ds(g * 16, 16)]
            return tuple(out)

        accs = lax.fori_loop(0, C // 8, red, (zero,) * 8)
        for g in range(4):
            tot = accs[g * 2] + accs[g * 2 + 1]
            q_v[i, pl.ds(g * 16, 16)] = tot * inv_c + mrows_v[i, pl.ds(g * 16, 16)]
        pend = nxt

    pltpu.sync_copy(q_v, out_hbm.at[pl.ds(base, NB)])


def _build_query(ctx_flat, missing_id, table):
    mesh = plsc.VectorSubcoreMesh(core_axis_name="c", subcore_axis_name="s")
    return pl.kernel(
        _query_body,
        out_type=jax.ShapeDtypeStruct((B, D), jnp.float32),
        mesh=mesh,
        scratch_types=[
            pltpu.VMEM((IDX_PER_W,), jnp.int32),
            pltpu.VMEM((C, D), jnp.float32),
            pltpu.VMEM((C, D), jnp.float32),
            pltpu.VMEM((NB,), jnp.int32),
            pltpu.VMEM((NB, D), jnp.float32),
            pltpu.VMEM((NB, D), jnp.float32),
            pltpu.SemaphoreType.DMA((2,)),
            pltpu.SemaphoreType.DMA,
        ],
        compiler_params=pltpu.CompilerParams(use_tc_tiling_on_sc=False),
    )(ctx_flat, missing_id, table)


def _score_body(q_ref, t_ref, o_ref, en_ref):
    i = pl.program_id(0)

    @pl.when(i == 0)
    def _init():
        t = t_ref[...]
        en2 = jnp.sum(t * t, axis=0, keepdims=True)
        en_ref[...] = 1.0 / jnp.maximum(jnp.sqrt(en2), EPS)

    q = q_ref[...]
    qn = jnp.maximum(jnp.sqrt(jnp.sum(q * q, axis=1, keepdims=True)), EPS)
    qs = q / qn
    dots = lax.dot_general(qs, t_ref[...], (((1,), (0,)), ((), ())),
                           preferred_element_type=jnp.float32)
    o_ref[...] = dots * en_ref[...]


def kernel(context_ids, missing_id, table):
    ctx_flat = context_ids.reshape(-1).astype(jnp.int32)
    miss = missing_id.astype(jnp.int32)
    query = _build_query(ctx_flat, miss, table)
    table_t = table.T
    scores = pl.pallas_call(
        _score_body,
        grid=(B // TB,),
        in_specs=[
            pl.BlockSpec((TB, D), lambda i: (i, 0)),
            pl.BlockSpec((D, V), lambda i: (0, 0)),
        ],
        out_specs=pl.BlockSpec((TB, V), lambda i: (i, 0)),
        out_shape=jax.ShapeDtypeStruct((B, V), jnp.float32),
        scratch_shapes=[pltpu.VMEM((1, V), jnp.float32)],
        compiler_params=pltpu.CompilerParams(
            vmem_limit_bytes=62 * 1024 * 1024),
    )(query, table_t)
    return scores


# speedup vs baseline: 1.0174x; 1.0174x over previous
import jax
import jax.numpy as jnp
from jax import lax
from jax.experimental import pallas as pl
from jax.experimental.pallas import tpu as pltpu
from jax.experimental.pallas import tpu_sc as plsc

V = 100000
D = 64
B = 1024
C = 200

NC = 2
NS = 16
NW = NC * NS
NB = B // NW
IDX_PER_W = NB * C

TB = 32
EPS = 1e-8


def _query_body(ctx_hbm, miss_hbm, table_hbm, out_hbm,
                idx_v, rows0_v, rows1_v, rows2_v, rows3_v, midx_v, mrows_v, q_v,
                gsems, msem):
    c = lax.axis_index("c")
    s = lax.axis_index("s")
    w = s * NC + c
    base = w * NB

    pltpu.sync_copy(ctx_hbm.at[pl.ds(base * C, IDX_PER_W)], idx_v)
    pltpu.sync_copy(miss_hbm.at[pl.ds(base, NB)], midx_v)
    pltpu.async_copy(table_hbm.at[midx_v], mrows_v, msem).wait()

    inv_c = jnp.float32(1.0 / C)
    bufs = (rows0_v, rows1_v, rows2_v, rows3_v)
    nbuf = len(bufs)

    def fire(i):
        buf = bufs[i % nbuf]
        sem = gsems.at[i % nbuf]
        c0 = pltpu.async_copy(table_hbm.at[idx_v.at[pl.ds(i * C, 128)]],
                              buf.at[pl.ds(0, 128)], sem)
        c1 = pltpu.async_copy(table_hbm.at[idx_v.at[pl.ds(i * C + 128, C - 128)]],
                              buf.at[pl.ds(128, C - 128)], sem)
        return c0, c1

    pending = [fire(i) for i in range(nbuf - 1)]
    zero = jnp.zeros((16,), jnp.float32)
    for i in range(NB):
        if i + nbuf - 1 < NB:
            pending.append(fire(i + nbuf - 1))
        pend = pending.pop(0)
        pend[0].wait()
        pend[1].wait()
        buf = bufs[i % nbuf]

        def red(j, accs, buf=buf):
            out = list(accs)
            for u in range(8):
                r = 8 * j + u
                p = u % 2
                for g in range(4):
                    k = g * 2 + p
                    out[k] = out[k] + buf[r, pl.ds(g * 16, 16)]
            return tuple(out)

        accs = lax.fori_loop(0, C // 8, red, (zero,) * 8)
        for g in range(4):
            tot = accs[g * 2] + accs[g * 2 + 1]
            q_v[i, pl.ds(g * 16, 16)] = tot * inv_c + mrows_v[i, pl.ds(g * 16, 16)]

    pltpu.sync_copy(q_v, out_hbm.at[pl.ds(base, NB)])


def _build_query(ctx_flat, missing_id, table):
    mesh = plsc.VectorSubcoreMesh(core_axis_name="c", subcore_axis_name="s")
    return pl.kernel(
        _query_body,
        out_type=jax.ShapeDtypeStruct((B, D), jnp.float32),
        mesh=mesh,
        scratch_types=[
            pltpu.VMEM((IDX_PER_W,), jnp.int32),
            pltpu.VMEM((C, D), jnp.float32),
            pltpu.VMEM((C, D), jnp.float32),
            pltpu.VMEM((C, D), jnp.float32),
            pltpu.VMEM((C, D), jnp.float32),
            pltpu.VMEM((NB,), jnp.int32),
            pltpu.VMEM((NB, D), jnp.float32),
            pltpu.VMEM((NB, D), jnp.float32),
            pltpu.SemaphoreType.DMA((4,)),
            pltpu.SemaphoreType.DMA,
        ],
        compiler_params=pltpu.CompilerParams(use_tc_tiling_on_sc=False),
    )(ctx_flat, missing_id, table)


def _score_body(q_ref, t_ref, o_ref, en_ref):
    i = pl.program_id(0)

    @pl.when(i == 0)
    def _init():
        t = t_ref[...]
        en2 = jnp.sum(t * t, axis=0, keepdims=True)
        en_ref[...] = 1.0 / jnp.maximum(jnp.sqrt(en2), EPS)

    q = q_ref[...]
    qn = jnp.maximum(jnp.sqrt(jnp.sum(q * q, axis=1, keepdims=True)), EPS)
    qs = q / qn
    dots = lax.dot_general(qs, t_ref[...], (((1,), (0,)), ((), ())),
                           preferred_element_type=jnp.float32)
    o_ref[...] = dots * en_ref[...]


def kernel(context_ids, missing_id, table):
    ctx_flat = context_ids.reshape(-1).astype(jnp.int32)
    miss = missing_id.astype(jnp.int32)
    query = _build_query(ctx_flat, miss, table)
    table_t = table.T
    scores = pl.pallas_call(
        _score_body,
        grid=(B // TB,),
        in_specs=[
            pl.BlockSpec((TB, D), lambda i: (i, 0)),
            pl.BlockSpec((D, V), lambda i: (0, 0)),
        ],
        out_specs=pl.BlockSpec((TB, V), lambda i: (i, 0)),
        out_shape=jax.ShapeDtypeStruct((B, V), jnp.float32),
        scratch_shapes=[pltpu.VMEM((1, V), jnp.float32)],
        compiler_params=pltpu.CompilerParams(
            vmem_limit_bytes=62 * 1024 * 1024),
    )(query, table_t)
    return scores
